# baseline (speedup 1.0000x reference)
import functools

import jax
import jax.numpy as jnp
from jax import lax
from jax.experimental import pallas as pl
from jax.experimental.pallas import tpu as pltpu
from jax.experimental.pallas import tpu_sc as plsc

_NC = 2
_NS = 16
_NW = _NC * _NS
_LANES = 16
_DIM = 4
_BLK = 128
_CHUNK = 8192
_BIAS = 8388608.0


@functools.lru_cache(maxsize=None)
def _build_sc_call(n_rows: int):
    rows_per_w = n_rows // _NW
    n_chunks = rows_per_w // _CHUNK
    vecs = _CHUNK // _LANES

    mesh = plsc.VectorSubcoreMesh(
        core_axis_name="c", subcore_axis_name="s",
        num_cores=_NC, num_subcores=_NS)

    @functools.partial(
        pl.kernel,
        out_type=jax.ShapeDtypeStruct((n_rows,), jnp.int32),
        mesh=mesh,
        compiler_params=pltpu.CompilerParams(needs_layout_passes=False),
        scratch_types=[
            pltpu.VMEM((_LANES,), jnp.float32),
            pltpu.VMEM((_CHUNK * _DIM,), jnp.float32),
            pltpu.VMEM((_CHUNK * _DIM,), jnp.float32),
            pltpu.VMEM((_CHUNK,), jnp.int32),
            pltpu.VMEM((_CHUNK,), jnp.int32),
            pltpu.SemaphoreType.DMA,
            pltpu.SemaphoreType.DMA,
            pltpu.SemaphoreType.DMA,
            pltpu.SemaphoreType.DMA,
        ],
    )
    def body(z_hbm, t_hbm, out_hbm, tbuf, xb0, xb1, ob0, ob1,
             is0, is1, os0, os1):
        wid = lax.axis_index("s") * _NC + lax.axis_index("c")
        pltpu.sync_copy(t_hbm, tbuf)

        zero = jnp.zeros((_LANES,), jnp.int32)
        t0 = plsc.load_gather(tbuf, [zero + 15])
        t14 = plsc.load_gather(tbuf, [zero + 14])
        sv = jnp.float32(14.0) / (t14 - t0)
        bv = jnp.float32(_BIAS) - t0 * sv
        lo = jnp.full((_LANES,), jnp.float32(_BIAS))
        hi = jnp.full((_LANES,), jnp.float32(_BIAS + 14.0))

        row0 = wid * rows_per_w
        bufs = ((xb0, ob0, is0, os0), (xb1, ob1, is1, os1))

        def in_slice(g):
            return z_hbm.at[pl.ds((row0 + g * _CHUNK) * _DIM, _CHUNK * _DIM)]

        def out_slice(g):
            return out_hbm.at[pl.ds(row0 + g * _CHUNK, _CHUNK)]

        def compute(xb, ob):
            @plsc.parallel_loop(0, vecs, unroll=16)
            def _loop(v):
                off = (v >> 3) * (_BLK * _DIM) + (v & 7) * _LANES
                terms = []
                for d in (3, 2, 1, 0):
                    xd = xb[pl.ds(off + d * _BLK, _LANES)]
                    y = jnp.minimum(jnp.maximum(xd * sv + bv, lo), hi)
                    k = plsc.bitcast(y, jnp.int32) & 0xF
                    q = plsc.load_gather(tbuf, [k])
                    bk = k + (xd > q).astype(jnp.int32)
                    terms.append(bk if d == 0 else bk << (4 * d))
                ob[pl.ds(v * _LANES, _LANES)] = (
                    (terms[0] + terms[1]) + (terms[2] + terms[3]))

        pltpu.async_copy(in_slice(0), xb0, is0)
        pltpu.async_copy(in_slice(1), xb1, is1)

        def outer(gg, _):
            for b, (xb, ob, isem, osem) in enumerate(bufs):
                g = gg * 2 + b
                pltpu.make_async_copy(in_slice(g), xb, isem).wait()

                @pl.when(gg >= 1)
                def _():
                    pltpu.make_async_copy(ob, out_slice(g - 2), osem).wait()

                compute(xb, ob)

                @pl.when(gg + 1 < n_chunks // 2)
                def _():
                    pltpu.async_copy(in_slice(g + 2), xb, isem)

                pltpu.async_copy(ob, out_slice(g), osem)
            return 0

        lax.fori_loop(0, n_chunks // 2, outer, 0)
        pltpu.make_async_copy(ob0, out_slice(n_chunks - 2), os0).wait()
        pltpu.make_async_copy(ob1, out_slice(n_chunks - 1), os1).wait()

    return body


def kernel(x, thresholds):
    n_rows = x.shape[0]
    z = x.reshape(n_rows // _BLK, _BLK, _DIM).transpose(0, 2, 1).reshape(-1)
    t16 = jnp.concatenate([thresholds, thresholds[:1]])
    out = _build_sc_call(n_rows)(z, t16)
    return out.astype(jnp.int64)


# speedup vs baseline: 2.3240x; 2.3240x over previous
import functools

import jax
import jax.numpy as jnp
from jax import lax
from jax.experimental import pallas as pl
from jax.experimental.pallas import tpu as pltpu
from jax.experimental.pallas import tpu_sc as plsc

_NC = 2
_NS = 16
_NW = _NC * _NS
_LANES = 16
_DIM = 4
_BLK = 128
_CHUNK = 8192
_BIAS = 8388608.0


@functools.lru_cache(maxsize=None)
def _build_sc_call(n_rows: int):
    rows_per_w = n_rows // _NW
    n_chunks = rows_per_w // _CHUNK
    vecs = _CHUNK // _LANES

    mesh = plsc.VectorSubcoreMesh(
        core_axis_name="c", subcore_axis_name="s",
        num_cores=_NC, num_subcores=_NS)

    @functools.partial(
        pl.kernel,
        out_type=jax.ShapeDtypeStruct((n_rows,), jnp.int32),
        mesh=mesh,
        compiler_params=pltpu.CompilerParams(needs_layout_passes=False),
        scratch_types=[
            pltpu.VMEM((_LANES,), jnp.float32),
            pltpu.VMEM((_CHUNK * _DIM,), jnp.float32),
            pltpu.VMEM((_CHUNK * _DIM,), jnp.float32),
            pltpu.VMEM((_CHUNK,), jnp.int32),
            pltpu.VMEM((_CHUNK,), jnp.int32),
            pltpu.SemaphoreType.DMA,
            pltpu.SemaphoreType.DMA,
            pltpu.SemaphoreType.DMA,
            pltpu.SemaphoreType.DMA,
        ],
    )
    def body(z_hbm, t_hbm, out_hbm, tbuf, xb0, xb1, ob0, ob1,
             is0, is1, os0, os1):
        wid = lax.axis_index("s") * _NC + lax.axis_index("c")
        pltpu.sync_copy(t_hbm, tbuf)

        zero = jnp.zeros((_LANES,), jnp.int32)
        t0 = plsc.load_gather(tbuf, [zero + 15])
        t14 = plsc.load_gather(tbuf, [zero + 14])
        sv = jnp.float32(14.0) / (t14 - t0)
        bv = jnp.float32(_BIAS) - t0 * sv
        lo = jnp.full((_LANES,), jnp.float32(_BIAS))
        hi = jnp.full((_LANES,), jnp.float32(_BIAS + 14.0))

        row0 = wid * rows_per_w
        bufs = ((xb0, ob0, is0, os0), (xb1, ob1, is1, os1))

        def in_slice(g):
            return z_hbm.at[pl.ds((row0 + g * _CHUNK) * _DIM, _CHUNK * _DIM)]

        def out_slice(g):
            return out_hbm.at[pl.ds(row0 + g * _CHUNK, _CHUNK)]

        def compute(xb, ob):
            @plsc.parallel_loop(0, vecs, unroll=8)
            def _loop(v):
                off = (v >> 3) * (_BLK * _DIM) + (v & 7) * _LANES
                terms = []
                for d in (3, 2, 1, 0):
                    xd = xb[pl.ds(off + d * _BLK, _LANES)]
                    y = jnp.minimum(jnp.maximum(xd * sv + bv, lo), hi)
                    k = plsc.bitcast(y, jnp.int32) & 0xF
                    q = plsc.load_gather(tbuf, [k])
                    bk = jnp.where(xd > q, k + 1, k)
                    terms.append(bk if d == 0 else bk << (4 * d))
                ob[pl.ds(v * _LANES, _LANES)] = (
                    (terms[0] + terms[1]) + (terms[2] + terms[3]))

        pltpu.async_copy(in_slice(0), xb0, is0)
        pltpu.async_copy(in_slice(1), xb1, is1)

        def outer(gg, _):
            for b, (xb, ob, isem, osem) in enumerate(bufs):
                g = gg * 2 + b
                pltpu.make_async_copy(in_slice(g), xb, isem).wait()

                @pl.when(gg >= 1)
                def _():
                    pltpu.make_async_copy(ob, out_slice(g - 2), osem).wait()

                compute(xb, ob)

                @pl.when(gg + 1 < n_chunks // 2)
                def _():
                    pltpu.async_copy(in_slice(g + 2), xb, isem)

                pltpu.async_copy(ob, out_slice(g), osem)
            return 0

        lax.fori_loop(0, n_chunks // 2, outer, 0)
        pltpu.make_async_copy(ob0, out_slice(n_chunks - 2), os0).wait()
        pltpu.make_async_copy(ob1, out_slice(n_chunks - 1), os1).wait()

    return body


def kernel(x, thresholds):
    n_rows = x.shape[0]
    z = x.reshape(n_rows // _BLK, _BLK, _DIM).transpose(0, 2, 1).reshape(-1)
    t16 = jnp.concatenate([thresholds, thresholds[:1]])
    out = _build_sc_call(n_rows)(z, t16)
    return out.astype(jnp.int64)
